# baseline (speedup 1.0000x reference)
import functools

import jax
import jax.numpy as jnp
from jax import lax
from jax.experimental import pallas as pl
from jax.experimental.pallas import tpu as pltpu
from jax.experimental.pallas import tpu_sc as plsc

B = 1024
IN = 128
OUT = 128
E = 64
CP = 128

NC = 2
NS = 16
NW = NC * NS
B_PER_W = B // NW
C_PER_W = B_PER_W * CP

E_BLK = 8
N_STEPS = E // E_BLK


def _sc_routing_body(gt_ref, c_ref, gvm, cvm):
    wid = lax.axis_index("s") * NC + lax.axis_index("c")
    base = wid * B_PER_W

    pltpu.sync_copy(gt_ref.at[pl.ds(base * 3, B_PER_W * 3)], gvm)

    def zero_body(i, carry):
        for u in range(8):
            cvm[pl.ds(i * 128 + u * 16, 16)] = jnp.zeros((16,), jnp.float32)
        return carry

    lax.fori_loop(0, C_PER_W // 128, zero_body, 0)

    for k in range(B_PER_W // 16):
        tl = lax.iota(jnp.int32, 16) + k * 16
        g = [plsc.load_gather(gvm, [tl * 3 + j]) for j in range(3)]
        one = jnp.float32(1.0)
        zero = jnp.float32(0.0)
        cnt = sum(jnp.where(gj != 0, one, zero) for gj in g)
        recip = 1.0 / jnp.maximum(cnt, 1.0)
        for j in range(3):
            val = jnp.where(g[j] != 0, recip, zero)
            plsc.addupdate_scatter(cvm, [tl * CP + g[j]], val)

    pltpu.sync_copy(cvm, c_ref.at[pl.ds(base * CP, C_PER_W)])


def _sc_routing(genres_t):
    mesh = plsc.VectorSubcoreMesh(core_axis_name="c", subcore_axis_name="s")
    run = pl.kernel(
        _sc_routing_body,
        out_type=jax.ShapeDtypeStruct((B * CP,), jnp.float32),
        mesh=mesh,
        scratch_types=[
            pltpu.VMEM((B_PER_W * 3,), jnp.int32),
            pltpu.VMEM((C_PER_W,), jnp.float32),
        ],
        compiler_params=pltpu.CompilerParams(needs_layout_passes=False),
    )
    return run(genres_t)


N_STEPS = 8
E_BLK = E // N_STEPS


def _tc_matmul_body(x_ref, c_ref, w_ref, out_ref, xs_ref, wb_ref, acc_ref):
    i = pl.program_id(0)

    @pl.when(i == 0)
    def _():
        x = x_ref[...].astype(jnp.bfloat16)
        c = c_ref[...].astype(jnp.bfloat16)
        for e in range(E):
            col = c[:, e : e + 1]
            xs_ref[e // E_BLK, :, (e % E_BLK) * IN : (e % E_BLK + 1) * IN] = (
                x * col
            )

    wb_ref[...] = w_ref[...].astype(jnp.bfloat16)
    part = lax.dot_general(
        xs_ref[i],
        wb_ref[...],
        (((1,), (0,)), ((), ())),
        preferred_element_type=jnp.float32,
    )

    @pl.when(i == 0)
    def _():
        acc_ref[...] = part

    @pl.when(i > 0)
    def _():
        acc_ref[...] += part

    @pl.when(i == N_STEPS - 1)
    def _():
        v = acc_ref[...]
        out_ref[...] = jnp.where(v >= 0.0, v, 0.01 * v)


def _tc_combine_matmul(x, c_mat, w_flat):
    return pl.pallas_call(
        _tc_matmul_body,
        grid=(N_STEPS,),
        in_specs=[
            pl.BlockSpec((B, IN), lambda i: (0, 0)),
            pl.BlockSpec((B, CP), lambda i: (0, 0)),
            pl.BlockSpec((E_BLK * IN, OUT), lambda i: (i, 0)),
        ],
        out_specs=pl.BlockSpec((B, OUT), lambda i: (0, 0)),
        out_shape=jax.ShapeDtypeStruct((B, OUT), jnp.float32),
        scratch_shapes=[
            pltpu.VMEM((N_STEPS, B, E_BLK * IN), jnp.bfloat16),
            pltpu.VMEM((E_BLK * IN, OUT), jnp.bfloat16),
            pltpu.VMEM((B, OUT), jnp.float32),
        ],
    )(x, c_mat, w_flat)


@jax.jit
def kernel(x, genres, W0):
    genres_t = genres.astype(jnp.int32).reshape(B * 3)
    w_flat = W0.reshape(E * IN, OUT)

    c_mat = _sc_routing(genres_t).reshape(B, CP)
    return _tc_combine_matmul(x, c_mat, w_flat)


# speedup vs baseline: 1.0841x; 1.0841x over previous
import functools

import jax
import jax.numpy as jnp
from jax import lax
from jax.experimental import pallas as pl
from jax.experimental.pallas import tpu as pltpu
from jax.experimental.pallas import tpu_sc as plsc

B = 1024
IN = 128
OUT = 128
E = 64
CP = 128

NC = 2
NS = 16
NW = NC * NS
B_PER_W = B // NW
C_PER_W = B_PER_W * CP

E_BLK = 8
N_STEPS = E // E_BLK


def _sc_routing_body(gt_ref, c_ref, gvm, cvm):
    wid = lax.axis_index("s") * NC + lax.axis_index("c")
    base = wid * B_PER_W

    for j in range(3):
        pltpu.sync_copy(gt_ref.at[pl.ds(j * B + base, B_PER_W)], gvm.at[j])

    def zero_body(i, carry):
        for u in range(8):
            cvm[pl.ds(i * 128 + u * 16, 16)] = jnp.zeros((16,), jnp.float32)
        return carry

    lax.fori_loop(0, C_PER_W // 128, zero_body, 0)

    for k in range(B_PER_W // 16):
        tl = lax.iota(jnp.int32, 16) + k * 16
        g = [gvm[j, pl.ds(k * 16, 16)] for j in range(3)]
        one = jnp.float32(1.0)
        zero = jnp.float32(0.0)
        cnt = sum(jnp.where(gj != 0, one, zero) for gj in g)
        recip = 1.0 / jnp.maximum(cnt, 1.0)
        for j in range(3):
            val = jnp.where(g[j] != 0, recip, zero)
            plsc.addupdate_scatter(cvm, [tl * CP + g[j]], val)

    pltpu.sync_copy(cvm, c_ref.at[pl.ds(base * CP, C_PER_W)])


def _sc_routing(genres_t):
    mesh = plsc.VectorSubcoreMesh(core_axis_name="c", subcore_axis_name="s")
    run = pl.kernel(
        _sc_routing_body,
        out_type=jax.ShapeDtypeStruct((B * CP,), jnp.float32),
        mesh=mesh,
        scratch_types=[
            pltpu.VMEM((3, B_PER_W), jnp.int32),
            pltpu.VMEM((C_PER_W,), jnp.float32),
        ],
        compiler_params=pltpu.CompilerParams(needs_layout_passes=False),
    )
    return run(genres_t)


def _tc_matmul_body(x_ref, c_ref, w_ref, out_ref, xs_ref, wb_ref):
    x = x_ref[...].astype(jnp.bfloat16)
    c = c_ref[...].astype(jnp.bfloat16)
    for e in range(E):
        col = c[:, e : e + 1]
        xs_ref[:, e * IN : (e + 1) * IN] = x * col
    wb_ref[...] = w_ref[...].astype(jnp.bfloat16)
    acc = lax.dot_general(
        xs_ref[...],
        wb_ref[...],
        (((1,), (0,)), ((), ())),
        preferred_element_type=jnp.float32,
    )
    out_ref[...] = jnp.where(acc >= 0.0, acc, 0.01 * acc)


def _tc_combine_matmul(x, c_mat, w_flat):
    return pl.pallas_call(
        _tc_matmul_body,
        grid=(1,),
        in_specs=[
            pl.BlockSpec((B, IN), lambda i: (0, 0)),
            pl.BlockSpec((B, CP), lambda i: (0, 0)),
            pl.BlockSpec((E * IN, OUT), lambda i: (0, 0)),
        ],
        out_specs=pl.BlockSpec((B, OUT), lambda i: (0, 0)),
        out_shape=jax.ShapeDtypeStruct((B, OUT), jnp.float32),
        scratch_shapes=[
            pltpu.VMEM((B, E * IN), jnp.bfloat16),
            pltpu.VMEM((E * IN, OUT), jnp.bfloat16),
        ],
    )(x, c_mat, w_flat)


@jax.jit
def kernel(x, genres, W0):
    genres_t = genres.astype(jnp.int32).T.reshape(3 * B)
    w_flat = W0.reshape(E * IN, OUT)

    c_mat = _sc_routing(genres_t).reshape(B, CP)
    return _tc_combine_matmul(x, c_mat, w_flat)


# speedup vs baseline: 1.1136x; 1.0272x over previous
import jax
import jax.numpy as jnp
from jax import lax
from jax.experimental import pallas as pl
from jax.experimental.pallas import tpu as pltpu
from jax.experimental.pallas import tpu_sc as plsc

B = 1024
IN = 128
OUT = 128
E = 64
CP = 128

NC = 2
NS = 16
NW = NC * NS
B_PER_W = B // NW
C_PER_W = B_PER_W * CP


def _sc_routing_body(gt_ref, c_ref, gvm, cvm):
    wid = lax.axis_index("s") * NC + lax.axis_index("c")
    base = wid * B_PER_W

    pltpu.sync_copy(gt_ref.at[pl.ds(base * 3, B_PER_W * 3)], gvm)

    def zero_body(i, carry):
        for u in range(8):
            cvm[pl.ds(i * 128 + u * 16, 16)] = jnp.zeros((16,), jnp.float32)
        return carry

    lax.fori_loop(0, C_PER_W // 128, zero_body, 0)

    for k in range(B_PER_W // 16):
        tl = lax.iota(jnp.int32, 16) + k * 16
        g = [plsc.load_gather(gvm, [tl * 3 + j]) for j in range(3)]
        one = jnp.float32(1.0)
        zero = jnp.float32(0.0)
        cnt = sum(jnp.where(gj != 0, one, zero) for gj in g)
        recip = 1.0 / jnp.maximum(cnt, 1.0)
        for j in range(3):
            val = jnp.where(g[j] != 0, recip, zero)
            plsc.addupdate_scatter(cvm, [tl * CP + g[j]], val)

    pltpu.sync_copy(cvm, c_ref.at[pl.ds(base * CP, C_PER_W)])


def _sc_routing(genres_t):
    mesh = plsc.VectorSubcoreMesh(core_axis_name="c", subcore_axis_name="s")
    run = pl.kernel(
        _sc_routing_body,
        out_type=jax.ShapeDtypeStruct((B * CP,), jnp.float32),
        mesh=mesh,
        scratch_types=[
            pltpu.VMEM((B_PER_W * 3,), jnp.int32),
            pltpu.VMEM((C_PER_W,), jnp.float32),
        ],
        compiler_params=pltpu.CompilerParams(needs_layout_passes=False),
    )
    return run(genres_t)


def _tc_matmul_body(x_ref, c_ref, w_ref, out_ref, xs_ref, wb_ref):
    x = x_ref[...].astype(jnp.bfloat16)
    c = c_ref[...].astype(jnp.bfloat16)
    for e in range(E):
        col = c[:, e : e + 1]
        xs_ref[:, e * IN : (e + 1) * IN] = x * col
    wb_ref[...] = w_ref[...].astype(jnp.bfloat16)
    acc = lax.dot_general(
        xs_ref[...],
        wb_ref[...],
        (((1,), (0,)), ((), ())),
        preferred_element_type=jnp.float32,
    )
    out_ref[...] = jnp.where(acc >= 0.0, acc, 0.01 * acc)


def _tc_combine_matmul(x, c_mat, w_flat):
    return pl.pallas_call(
        _tc_matmul_body,
        grid=(1,),
        in_specs=[
            pl.BlockSpec((B, IN), lambda i: (0, 0)),
            pl.BlockSpec((B, CP), lambda i: (0, 0)),
            pl.BlockSpec((E * IN, OUT), lambda i: (0, 0)),
        ],
        out_specs=pl.BlockSpec((B, OUT), lambda i: (0, 0)),
        out_shape=jax.ShapeDtypeStruct((B, OUT), jnp.float32),
        scratch_shapes=[
            pltpu.VMEM((B, E * IN), jnp.bfloat16),
            pltpu.VMEM((E * IN, OUT), jnp.bfloat16),
        ],
    )(x, c_mat, w_flat)


@jax.jit
def kernel(x, genres, W0):
    genres_t = genres.astype(jnp.int32).reshape(B * 3)
    w_flat = W0.reshape(E * IN, OUT)

    c_mat = _sc_routing(genres_t).reshape(B, CP)
    return _tc_combine_matmul(x, c_mat, w_flat)
